# baseline (speedup 1.0000x reference)
import jax
import jax.numpy as jnp
from jax import lax
from jax.experimental import pallas as pl
from jax.experimental.pallas import tpu as pltpu
from jax.experimental.pallas import tpu_sc as plsc

N_CLASSES = 1000000
EMBED_DIM = 32
BATCH = 16384

NC = 2
NS = 16
NW = NC * NS
B_PER_W = BATCH // NW
LANES = 128
RCP = 8
SUBS = 8
MACROS = B_PER_W // (RCP * SUBS)


def _gather_body(idx_hbm, tableT_hbm, out_hbm, idx_v, buf_a, buf_b,
                 out_stage, sem):
    wid = lax.axis_index("s") * NC + lax.axis_index("c")
    base = wid * B_PER_W
    pltpu.sync_copy(idx_hbm.at[pl.ds(base, B_PER_W)], idx_v)

    lanes16 = lax.iota(jnp.int32, 16)
    bufs = (buf_a, buf_b)

    def load_rv(m, local):
        return idx_v[pl.ds((m * (SUBS // 2) + local // 2) * 16, 16)]

    def fire(m, local):
        rv = load_rv(m, local)
        buf = bufs[local % 2]
        for k in range(RCP):
            lane = (local % 2) * RCP + k
            start = pl.multiple_of((rv[lane] >> 7) << 7, LANES)
            pltpu.async_copy(
                tableT_hbm.at[:, pl.ds(start, LANES)], buf.at[k], sem
            )

    def drain(local):
        buf = bufs[local % 2]
        for k in range(RCP):
            pltpu.make_async_copy(
                tableT_hbm.at[:, pl.ds(0, LANES)], buf.at[k], sem
            ).wait()

    def extract(m, local):
        rv = load_rv(m, local)
        buf = bufs[local % 2]
        for k in range(RCP):
            lane = (local % 2) * RCP + k
            lane_vec = jnp.full((16,), rv[lane] & 127, jnp.int32)
            kf = jnp.full((16,), k, jnp.int32)
            out_stage[k, pl.ds(0, 16)] = plsc.load_gather(
                buf, [kf, lanes16, lane_vec])
            out_stage[k, pl.ds(16, 16)] = plsc.load_gather(
                buf, [kf, lanes16 + 16, lane_vec])
        row0 = pl.multiple_of(base + (m * SUBS + local) * RCP, RCP)
        pltpu.sync_copy(out_stage, out_hbm.at[pl.ds(row0, RCP)])

    fire(0, 0)

    def macro_body(m, _):
        for local in range(SUBS):
            if local < SUBS - 1:
                fire(m, local + 1)
            else:
                @pl.when(m < MACROS - 1)
                def _():
                    fire(m + 1, 0)
            drain(local)
            extract(m, local)
        return ()

    lax.fori_loop(0, MACROS, macro_body, ())
    return


@jax.jit
def kernel(batch, table):
    mesh = plsc.VectorSubcoreMesh(
        core_axis_name="c", subcore_axis_name="s",
        num_cores=NC, num_subcores=NS,
    )
    out = pl.kernel(
        _gather_body,
        out_type=jax.ShapeDtypeStruct((BATCH, EMBED_DIM), jnp.float32),
        mesh=mesh,
        scratch_types=[
            pltpu.VMEM((B_PER_W,), jnp.int32),
            pltpu.VMEM((RCP, EMBED_DIM, LANES), jnp.float32),
            pltpu.VMEM((RCP, EMBED_DIM, LANES), jnp.float32),
            pltpu.VMEM((RCP, EMBED_DIM), jnp.float32),
            pltpu.SemaphoreType.DMA,
        ],
        compiler_params=pltpu.CompilerParams(needs_layout_passes=False),
    )(batch, table.T)
    return out.reshape(BATCH, 1, EMBED_DIM)
